# baseline (speedup 1.0000x reference)
import functools

import jax
import jax.numpy as jnp
from jax.experimental import pallas as pl
from jax.experimental.pallas import tpu as pltpu


def _cdiv(a, b):
    return (a + b - 1) // b


def _mlp_sigmoid_sum_kernel(x_ref, w1_ref, b1_ref, w2_ref, ones_ref, b2_ref,
                            out_ref, *, tm, tc, m_total, masked):
    w1 = w1_ref[...]
    b1 = b1_ref[...]
    w2 = w2_ref[...]
    ones8 = ones_ref[...]
    b2 = b2_ref[0]

    n_chunks = tm // tc

    def fc1(c):
        xq = x_ref[pl.ds(c * tc, tc), :].astype(jnp.float8_e4m3fn)
        return jnp.dot(xq, w1, preferred_element_type=jnp.float32)

    def tail(c, h):
        hb = jnp.maximum(h.astype(jnp.bfloat16), b1)
        hw = hb * w2
        p = (hw[:, 0:128] + hw[:, 128:256]) + (hw[:, 256:384] + hw[:, 384:512])
        yt = jax.lax.dot_general(
            ones8, p,
            dimension_numbers=(((1,), (1,)), ((), ())),
            preferred_element_type=jnp.float32)
        sig = jax.nn.sigmoid(yt + b2)
        if masked:
            i = pl.program_id(0)
            lane = jax.lax.broadcasted_iota(jnp.int32, sig.shape, 1)
            sig = jnp.where(i * tm + c * tc + lane < m_total, sig, 0.0)
        return sum(sig[:, j * 128:(j + 1) * 128] for j in range(tc // 128))

    partial = jnp.zeros((8, 128), jnp.float32)
    h_prev = fc1(0)
    for c in range(1, n_chunks):
        h_cur = fc1(c)
        partial = partial + tail(c - 1, h_prev)
        h_prev = h_cur
    partial = partial + tail(n_chunks - 1, h_prev)

    out_ref[...] = partial[None]


def _pick_tm(m):
    for tm in (32768, 16384, 8192, 4096, 2048, 1024, 512):
        if m % tm == 0:
            return tm, False
    return min(8192, _cdiv(m, 1024) * 1024), True


def kernel(x, w1, b1, w2, b2):
    d = x.shape[-1]
    x2d = x.reshape(-1, d).astype(jnp.float32)
    m, _ = x2d.shape
    h_dim = w1.shape[0]

    tm, masked = _pick_tm(m)
    m_pad = _cdiv(m, tm) * tm
    if m_pad != m:
        x2d = jnp.pad(x2d, ((0, m_pad - m), (0, 0)))
    num_blocks = m_pad // tm

    w1_bf = jnp.asarray(w1, jnp.float32).T.astype(jnp.float8_e4m3fn)
    b1_bf = jnp.asarray(b1, jnp.float32).reshape(1, h_dim).astype(jnp.bfloat16)
    w2_bf = jnp.asarray(w2, jnp.float32).reshape(1, h_dim).astype(jnp.bfloat16)
    b1_neg = -b1_bf
    ones8 = jnp.ones((8, 128), jnp.bfloat16)
    b2_s = (jnp.asarray(b2, jnp.float32).reshape(1)
            + jnp.sum(w2_bf.astype(jnp.float32) * b1_bf.astype(jnp.float32),
                      axis=1))

    tc = min(tm, 1024)
    body = functools.partial(_mlp_sigmoid_sum_kernel,
                             tm=tm, tc=tc, m_total=m, masked=masked)

    partials = pl.pallas_call(
        body,
        out_shape=jax.ShapeDtypeStruct((num_blocks, 8, 128), jnp.float32),
        grid=(num_blocks,),
        in_specs=[
            pl.BlockSpec((tm, d), lambda i: (i, 0)),
            pl.BlockSpec((d, h_dim), lambda i: (0, 0)),
            pl.BlockSpec((1, h_dim), lambda i: (0, 0)),
            pl.BlockSpec((1, h_dim), lambda i: (0, 0)),
            pl.BlockSpec((8, 128), lambda i: (0, 0)),
            pl.BlockSpec(memory_space=pltpu.MemorySpace.SMEM),
        ],
        out_specs=pl.BlockSpec((1, 8, 128), lambda i: (i, 0, 0)),
        compiler_params=pltpu.CompilerParams(
            dimension_semantics=("parallel",),
            vmem_limit_bytes=100 * 1024 * 1024,
        ),
    )(x2d, w1_bf, b1_neg, w2_bf, ones8, b2_s)

    return 0.5 + 0.5 * (jnp.sum(partials) * 0.125) / m


# speedup vs baseline: 1.0082x; 1.0082x over previous
import functools

import jax
import jax.numpy as jnp
from jax.experimental import pallas as pl
from jax.experimental.pallas import tpu as pltpu


def _cdiv(a, b):
    return (a + b - 1) // b


def _mlp_sigmoid_sum_kernel(xa_ref, xb_ref, w1_ref, b1_ref, w2_ref, ones_ref,
                            b2_ref, out_ref, *, tm, tc, m_total, masked):
    w1 = w1_ref[...]
    b1 = b1_ref[...]
    w2 = w2_ref[...]
    ones8 = ones_ref[...]
    b2 = b2_ref[0]

    half = tm // 2
    n_chunks = tm // tc

    def fc1(c):
        base = c * tc
        ref = xa_ref if base < half else xb_ref
        xq = ref[pl.ds(base % half, tc), :].astype(jnp.float8_e4m3fn)
        return jnp.dot(xq, w1, preferred_element_type=jnp.float32)

    def tail(c, h):
        hb = jnp.maximum(h.astype(jnp.bfloat16), b1)
        hw = hb * w2
        p = (hw[:, 0:128] + hw[:, 128:256]) + (hw[:, 256:384] + hw[:, 384:512])
        yt = jax.lax.dot_general(
            ones8, p,
            dimension_numbers=(((1,), (1,)), ((), ())),
            preferred_element_type=jnp.float32)
        sig = jax.nn.sigmoid(yt + b2)
        if masked:
            i = pl.program_id(0)
            lane = jax.lax.broadcasted_iota(jnp.int32, sig.shape, 1)
            sig = jnp.where(i * tm + c * tc + lane < m_total, sig, 0.0)
        return sum(sig[:, j * 128:(j + 1) * 128] for j in range(tc // 128))

    partial = jnp.zeros((8, 128), jnp.float32)
    h_prev = fc1(0)
    for c in range(1, n_chunks):
        h_cur = fc1(c)
        partial = partial + tail(c - 1, h_prev)
        h_prev = h_cur
    partial = partial + tail(n_chunks - 1, h_prev)

    out_ref[...] = partial[None]


def _pick_tm(m):
    for tm in (16384, 8192, 4096, 2048):
        if m % tm == 0:
            return tm, False
    return min(8192, _cdiv(m, 2048) * 2048), True


def kernel(x, w1, b1, w2, b2):
    d = x.shape[-1]
    x2d = x.reshape(-1, d).astype(jnp.float32)
    m, _ = x2d.shape
    h_dim = w1.shape[0]

    tm, masked = _pick_tm(m)
    m_pad = _cdiv(m, tm) * tm
    if m_pad != m:
        x2d = jnp.pad(x2d, ((0, m_pad - m), (0, 0)))
    num_blocks = m_pad // tm

    w1_bf = jnp.asarray(w1, jnp.float32).T.astype(jnp.float8_e4m3fn)
    b1_bf = jnp.asarray(b1, jnp.float32).reshape(1, h_dim).astype(jnp.bfloat16)
    w2_bf = jnp.asarray(w2, jnp.float32).reshape(1, h_dim).astype(jnp.bfloat16)
    b1_neg = -b1_bf
    ones8 = jnp.ones((8, 128), jnp.bfloat16)
    b2_s = (jnp.asarray(b2, jnp.float32).reshape(1)
            + jnp.sum(w2_bf.astype(jnp.float32) * b1_bf.astype(jnp.float32),
                      axis=1))

    tc = min(tm, 1024)
    body = functools.partial(_mlp_sigmoid_sum_kernel,
                             tm=tm, tc=tc, m_total=m, masked=masked)

    partials = pl.pallas_call(
        body,
        out_shape=jax.ShapeDtypeStruct((num_blocks, 8, 128), jnp.float32),
        grid=(num_blocks,),
        in_specs=[
            pl.BlockSpec((tm // 2, d), lambda i: (2 * i, 0)),
            pl.BlockSpec((tm // 2, d), lambda i: (2 * i + 1, 0)),
            pl.BlockSpec((d, h_dim), lambda i: (0, 0)),
            pl.BlockSpec((1, h_dim), lambda i: (0, 0)),
            pl.BlockSpec((1, h_dim), lambda i: (0, 0)),
            pl.BlockSpec((8, 128), lambda i: (0, 0)),
            pl.BlockSpec(memory_space=pltpu.MemorySpace.SMEM),
        ],
        out_specs=pl.BlockSpec((1, 8, 128), lambda i: (i, 0, 0)),
        compiler_params=pltpu.CompilerParams(
            dimension_semantics=("parallel",),
            vmem_limit_bytes=100 * 1024 * 1024,
        ),
    )(x2d, x2d, w1_bf, b1_neg, w2_bf, ones8, b2_s)

    return 0.5 + 0.5 * (jnp.sum(partials) * 0.125) / m


# speedup vs baseline: 1.0149x; 1.0067x over previous
import functools

import jax
import jax.numpy as jnp
from jax.experimental import pallas as pl
from jax.experimental.pallas import tpu as pltpu


def _cdiv(a, b):
    return (a + b - 1) // b


def _mlp_sigmoid_sum_kernel(x_ref, w1_ref, b1_ref, w2_ref, ones_ref,
                            b2_ref, out_ref, *, tm, tc, m_total, masked):
    w1 = w1_ref[...]
    b1 = b1_ref[...]
    w2 = w2_ref[...]
    ones8 = ones_ref[...]
    b2 = b2_ref[0]

    n_chunks = tm // tc

    def fc1(c):
        xq = x_ref[pl.ds(c * tc, tc), :].astype(jnp.float8_e4m3fn)
        return jnp.dot(xq, w1, preferred_element_type=jnp.float32)

    def tail(c, h):
        hb = jnp.maximum(h.astype(jnp.bfloat16), b1)
        hw = hb * w2
        p = (hw[:, 0:128] + hw[:, 128:256]) + (hw[:, 256:384] + hw[:, 384:512])
        yt = jax.lax.dot_general(
            ones8, p,
            dimension_numbers=(((1,), (1,)), ((), ())),
            preferred_element_type=jnp.float32)
        sig = jax.nn.sigmoid(yt + b2)
        if masked:
            i = pl.program_id(0)
            lane = jax.lax.broadcasted_iota(jnp.int32, sig.shape, 1)
            sig = jnp.where(i * tm + c * tc + lane < m_total, sig, 0.0)
        return sum(sig[:, j * 128:(j + 1) * 128] for j in range(tc // 128))

    partial = jnp.zeros((8, 128), jnp.float32)
    h_prev = fc1(0)
    for c in range(1, n_chunks):
        h_cur = fc1(c)
        partial = partial + tail(c - 1, h_prev)
        h_prev = h_cur
    partial = partial + tail(n_chunks - 1, h_prev)

    out_ref[...] = partial[None]


def _pick_tm(m):
    for tm in (16384, 8192, 4096, 2048, 1024, 512):
        if m % tm == 0:
            return tm, False
    return min(8192, _cdiv(m, 1024) * 1024), True


def kernel(x, w1, b1, w2, b2):
    d = x.shape[-1]
    x2d = x.reshape(-1, d).astype(jnp.float32)
    m, _ = x2d.shape
    h_dim = w1.shape[0]

    tm, masked = _pick_tm(m)
    m_pad = _cdiv(m, tm) * tm
    if m_pad != m:
        x2d = jnp.pad(x2d, ((0, m_pad - m), (0, 0)))
    num_blocks = m_pad // tm

    w1_bf = jnp.asarray(w1, jnp.float32).T.astype(jnp.float8_e4m3fn)
    b1_bf = jnp.asarray(b1, jnp.float32).reshape(1, h_dim).astype(jnp.bfloat16)
    w2_bf = jnp.asarray(w2, jnp.float32).reshape(1, h_dim).astype(jnp.bfloat16)
    b1_neg = -b1_bf
    ones8 = jnp.ones((8, 128), jnp.bfloat16)
    b2_s = (jnp.asarray(b2, jnp.float32).reshape(1)
            + jnp.sum(w2_bf.astype(jnp.float32) * b1_bf.astype(jnp.float32),
                      axis=1))

    tc = min(tm, 1024)
    body = functools.partial(_mlp_sigmoid_sum_kernel,
                             tm=tm, tc=tc, m_total=m, masked=masked)

    partials = pl.pallas_call(
        body,
        out_shape=jax.ShapeDtypeStruct((num_blocks, 8, 128), jnp.float32),
        grid=(num_blocks,),
        in_specs=[
            pl.BlockSpec((tm, d), lambda i: (i, 0)),
            pl.BlockSpec((d, h_dim), lambda i: (0, 0)),
            pl.BlockSpec((1, h_dim), lambda i: (0, 0)),
            pl.BlockSpec((1, h_dim), lambda i: (0, 0)),
            pl.BlockSpec((8, 128), lambda i: (0, 0)),
            pl.BlockSpec(memory_space=pltpu.MemorySpace.SMEM),
        ],
        out_specs=pl.BlockSpec((1, 8, 128), lambda i: (i, 0, 0)),
        compiler_params=pltpu.CompilerParams(
            dimension_semantics=("parallel",),
            vmem_limit_bytes=100 * 1024 * 1024,
        ),
    )(x2d, w1_bf, b1_neg, w2_bf, ones8, b2_s)

    return 0.5 + 0.5 * (jnp.sum(partials) * 0.125) / m
